# baseline (speedup 1.0000x reference)
import functools

import jax
import jax.numpy as jnp
from jax import lax
from jax.experimental import pallas as pl
from jax.experimental.pallas import tpu as pltpu
from jax.experimental.pallas import tpu_sc as plsc

H = 2048
E = 16
TOPK = 2
DFF = 512
DSH = 512
T = 2048
BM = 256
NPAD = T * TOPK + E * BM
NB = NPAD // BM
NC = 2
NS = 16
NW = NC * NS
LANES = 16


def _silu(x):
    return x * jax.nn.sigmoid(x)


def _router_body(x_ref, gw_ref, logits_ref, d_ref, w_ref, tok_ref, blk_ref,
                 act_ref):
    x = x_ref[...]
    gw = gw_ref[...]
    logits = lax.dot_general(x, gw, (((1,), (1,)), ((), ())),
                             preferred_element_type=jnp.float32)
    logits_ref[...] = logits
    m = jnp.max(logits, axis=-1, keepdims=True)
    ex = jnp.exp(logits - m)
    probs = ex / jnp.sum(ex, axis=-1, keepdims=True)
    iota_e = lax.broadcasted_iota(jnp.int32, (T, E), 1)
    m1 = jnp.max(probs, axis=-1, keepdims=True)
    i1 = jnp.min(jnp.where(probs == m1, iota_e, E), axis=-1, keepdims=True)
    oh1 = iota_e == i1
    pm = jnp.where(oh1, -jnp.inf, probs)
    m2 = jnp.max(pm, axis=-1, keepdims=True)
    i2 = jnp.min(jnp.where(pm == m2, iota_e, E), axis=-1, keepdims=True)
    oh2 = iota_e == i2
    s = m1 + m2
    w_ref[...] = jnp.concatenate([m1 / s, m2 / s], axis=1)
    a = (oh1 | oh2).astype(jnp.float32)
    it_r = lax.broadcasted_iota(jnp.int32, (T, T), 0)
    it_c = lax.broadcasted_iota(jnp.int32, (T, T), 1)
    ltri = (it_c < it_r).astype(jnp.float32)
    rank = lax.dot_general(ltri, a, (((1,), (0,)), ((), ())),
                           preferred_element_type=jnp.float32)
    cnt = jnp.sum(a, axis=0, keepdims=True)
    cpad = jnp.ceil(cnt / BM) * BM
    ie_r = lax.broadcasted_iota(jnp.int32, (E, E), 0)
    ie_c = lax.broadcasted_iota(jnp.int32, (E, E), 1)
    utri = (ie_r < ie_c).astype(jnp.float32)
    off = lax.dot_general(cpad, utri, (((1,), (0,)), ((), ())),
                          preferred_element_type=jnp.float32)
    dst = off + rank
    d0 = jnp.sum(jnp.where(oh1, dst, 0.0), axis=1, keepdims=True)
    d1 = jnp.sum(jnp.where(oh2, dst, 0.0), axis=1, keepdims=True)
    d_ref[...] = jnp.concatenate([d0, d1], axis=1).astype(jnp.int32)
    tok_ref[...] = lax.broadcasted_iota(jnp.int32, (T, TOPK), 0)
    ones_t = jnp.ones((T, 1), jnp.float32)
    cnt_c = lax.dot_general(a, ones_t, (((0,), (0,)), ((), ())),
                            preferred_element_type=jnp.float32)
    cpad_c = jnp.ceil(cnt_c / BM) * BM
    ltri_e = (ie_c < ie_r).astype(jnp.float32)
    off_c = lax.dot_general(ltri_e, cpad_c, (((1,), (0,)), ((), ())),
                            preferred_element_type=jnp.float32)
    bpos = (lax.broadcasted_iota(jnp.int32, (1, NB), 1) * BM).astype(
        jnp.float32)
    blk = jnp.sum((off_c <= bpos).astype(jnp.int32), axis=0, keepdims=True) - 1
    blk_ref[...] = blk
    total = jnp.sum(cpad, axis=1, keepdims=True)
    act_ref[...] = (bpos < total).astype(jnp.int32)


def _router(x, gate_w):
    return pl.pallas_call(
        _router_body,
        out_shape=[
            jax.ShapeDtypeStruct((T, E), jnp.float32),
            jax.ShapeDtypeStruct((T, TOPK), jnp.int32),
            jax.ShapeDtypeStruct((T, TOPK), jnp.float32),
            jax.ShapeDtypeStruct((T, TOPK), jnp.int32),
            jax.ShapeDtypeStruct((1, NB), jnp.int32),
            jax.ShapeDtypeStruct((1, NB), jnp.int32),
        ],
    )(x, gate_w)


def _dispatch_body(x_hbm, tok_hbm, d_hbm, xs_hbm, tokv, dv, rowsa, rowsb,
                   gsem, ssem):
    wid = lax.axis_index("s") * NC + lax.axis_index("c")
    pltpu.sync_copy(tok_hbm.at[wid], tokv)
    pltpu.sync_copy(d_hbm.at[wid], dv)
    nch = T * TOPK // NW // LANES
    bufs = [rowsa, rowsb]
    gcp = [None] * nch
    scp = [None] * nch
    gcp[0] = pltpu.async_copy(x_hbm.at[tokv.at[0]], bufs[0], gsem)
    for ci in range(nch):
        cur = bufs[ci % 2]
        gcp[ci].wait()
        scp[ci] = pltpu.async_copy(cur, xs_hbm.at[dv.at[ci]], ssem)
        if ci + 1 < nch:
            if ci >= 1:
                scp[ci - 1].wait()
            gcp[ci + 1] = pltpu.async_copy(
                x_hbm.at[tokv.at[ci + 1]], bufs[(ci + 1) % 2], gsem)
    scp[nch - 2].wait()
    scp[nch - 1].wait()


def _dispatch(x, tok, d):
    nch = T * TOPK // NW // LANES
    mesh = plsc.VectorSubcoreMesh(core_axis_name="c", subcore_axis_name="s")
    kern = functools.partial(
        pl.kernel,
        mesh=mesh,
        out_type=jax.ShapeDtypeStruct((NPAD, H), jnp.float32),
        scratch_types=[
            pltpu.VMEM((nch, LANES), jnp.int32),
            pltpu.VMEM((nch, LANES), jnp.int32),
            pltpu.VMEM((LANES, H), jnp.float32),
            pltpu.VMEM((LANES, H), jnp.float32),
            pltpu.SemaphoreType.DMA,
            pltpu.SemaphoreType.DMA,
        ],
    )(_dispatch_body)
    return kern(x, tok.reshape(NW, nch, LANES), d.reshape(NW, nch, LANES))


def _gmm_body(blk_ref, act_ref, xs_ref, x_ref, wg_ref, wu_ref, wd_ref,
              sg_ref, su_ref, sd_ref, sgw_ref, ys_ref, sh_ref):
    b = pl.program_id(0)

    @pl.when(jnp.logical_and(b < NB, act_ref[0, jnp.minimum(b, NB - 1)] == 1))
    def _():
        xb = xs_ref[...]
        hg = lax.dot_general(xb, wg_ref[0], (((1,), (1,)), ((), ())),
                             preferred_element_type=jnp.float32)
        hu = lax.dot_general(xb, wu_ref[0], (((1,), (1,)), ((), ())),
                             preferred_element_type=jnp.float32)
        hmid = _silu(hg) * hu
        ys_ref[...] = lax.dot_general(hmid, wd_ref[0],
                                      (((1,), (1,)), ((), ())),
                                      preferred_element_type=jnp.float32)

    @pl.when(b >= NB)
    def _():
        xb = x_ref[...]
        g = lax.dot_general(xb, sg_ref[...], (((1,), (1,)), ((), ())),
                            preferred_element_type=jnp.float32)
        u = lax.dot_general(xb, su_ref[...], (((1,), (1,)), ((), ())),
                            preferred_element_type=jnp.float32)
        mid = _silu(g) * u
        y = lax.dot_general(mid, sd_ref[...], (((1,), (1,)), ((), ())),
                            preferred_element_type=jnp.float32)
        gate = jax.nn.sigmoid(
            lax.dot_general(xb, sgw_ref[...], (((1,), (1,)), ((), ())),
                            preferred_element_type=jnp.float32))
        sh_ref[...] = gate * y


def _gmm_shared(blk_e, act, xs, x, Wg, Wu, Wd, Sg, Su, Sd, sgw):
    nsh = T // BM

    def _clamp(b):
        return jnp.minimum(b, NB - 1)

    grid_spec = pltpu.PrefetchScalarGridSpec(
        num_scalar_prefetch=2,
        grid=(NB + nsh,),
        in_specs=[
            pl.BlockSpec((BM, H), lambda b, blk, act: (_clamp(b), 0)),
            pl.BlockSpec((BM, H),
                         lambda b, blk, act: (jnp.maximum(b - NB, 0), 0)),
            pl.BlockSpec((1, DFF, H),
                         lambda b, blk, act: (blk[0, _clamp(b)], 0, 0)),
            pl.BlockSpec((1, DFF, H),
                         lambda b, blk, act: (blk[0, _clamp(b)], 0, 0)),
            pl.BlockSpec((1, H, DFF),
                         lambda b, blk, act: (blk[0, _clamp(b)], 0, 0)),
            pl.BlockSpec((DSH, H), lambda b, blk, act: (0, 0)),
            pl.BlockSpec((DSH, H), lambda b, blk, act: (0, 0)),
            pl.BlockSpec((H, DSH), lambda b, blk, act: (0, 0)),
            pl.BlockSpec((1, H), lambda b, blk, act: (0, 0)),
        ],
        out_specs=[
            pl.BlockSpec((BM, H), lambda b, blk, act: (_clamp(b), 0)),
            pl.BlockSpec((BM, H),
                         lambda b, blk, act: (jnp.maximum(b - NB, 0), 0)),
        ],
    )
    return pl.pallas_call(
        _gmm_body,
        grid_spec=grid_spec,
        out_shape=[
            jax.ShapeDtypeStruct((NPAD, H), jnp.float32),
            jax.ShapeDtypeStruct((T, H), jnp.float32),
        ],
    )(blk_e, act, xs, x, Wg, Wu, Wd, Sg, Su, Sd, sgw)


def _combine_body(ys_hbm, sh_hbm, d0_hbm, d1_hbm, w0_hbm, w1_hbm, out_hbm,
                  d0v, d1v, w0v, w1v, y0, y1, shv, sem):
    wid = lax.axis_index("s") * NC + lax.axis_index("c")
    tpw = T // NW
    nch = tpw // LANES
    base = wid * tpw
    pltpu.sync_copy(d0_hbm.at[wid], d0v)
    pltpu.sync_copy(d1_hbm.at[wid], d1v)
    pltpu.sync_copy(w0_hbm.at[wid], w0v)
    pltpu.sync_copy(w1_hbm.at[wid], w1v)
    for ci in range(nch):
        pltpu.async_copy(ys_hbm.at[d0v.at[ci]], y0, sem).wait()
        pltpu.async_copy(ys_hbm.at[d1v.at[ci]], y1, sem).wait()
        pltpu.sync_copy(sh_hbm.at[pl.ds(base + ci * LANES, LANES)], shv)
        w0c = w0v[ci]
        w1c = w1v[ci]
        dnums = lax.GatherDimensionNumbers(
            offset_dims=(), collapsed_slice_dims=(0,), start_index_map=(0,))
        for r in range(LANES):
            ridx = jnp.full((LANES, 1), r, jnp.int32)
            w0r = lax.gather(w0c, ridx, dnums, (1,),
                             mode=lax.GatherScatterMode.PROMISE_IN_BOUNDS)
            w1r = lax.gather(w1c, ridx, dnums, (1,),
                             mode=lax.GatherScatterMode.PROMISE_IN_BOUNDS)

            def col(cidx, carry, r=r, w0r=w0r, w1r=w1r):
                cs = pl.ds(cidx * LANES, LANES)
                shv[r, cs] = (w0r * y0[r, cs] + w1r * y1[r, cs] + shv[r, cs])
                return carry

            lax.fori_loop(0, H // LANES, col, 0)
        pltpu.sync_copy(shv, out_hbm.at[pl.ds(base + ci * LANES, LANES)])


def _combine(ys, sh, d, w):
    tpw = T // NW
    nch = tpw // LANES
    mesh = plsc.VectorSubcoreMesh(core_axis_name="c", subcore_axis_name="s")
    d0 = d[:, 0].reshape(NW, nch, LANES)
    d1 = d[:, 1].reshape(NW, nch, LANES)
    w0 = w[:, 0].reshape(NW, nch, LANES)
    w1 = w[:, 1].reshape(NW, nch, LANES)
    kern = functools.partial(
        pl.kernel,
        mesh=mesh,
        out_type=jax.ShapeDtypeStruct((T, H), jnp.float32),
        scratch_types=[
            pltpu.VMEM((nch, LANES), jnp.int32),
            pltpu.VMEM((nch, LANES), jnp.int32),
            pltpu.VMEM((nch, LANES), jnp.float32),
            pltpu.VMEM((nch, LANES), jnp.float32),
            pltpu.VMEM((LANES, H), jnp.float32),
            pltpu.VMEM((LANES, H), jnp.float32),
            pltpu.VMEM((LANES, H), jnp.float32),
            pltpu.SemaphoreType.DMA,
        ],
    )(_combine_body)
    return kern(ys, sh, d0, d1, w0, w1)


def kernel(hidden_states, gate_w, Wg, Wu, Wd, Sg, Su, Sd, shared_gate_w):
    b, s, h = hidden_states.shape
    x = hidden_states.reshape(-1, h)
    logits, d, w, tok, blk_e, act = _router(x, gate_w)
    xs = _dispatch(x, tok, d)
    ys, sh = _gmm_shared(blk_e, act, xs, x, Wg, Wu, Wd, Sg, Su, Sd,
                         shared_gate_w)
    final = _combine(ys, sh, d, w)
    return final.reshape(b, s, h), logits


# speedup vs baseline: 1.0356x; 1.0356x over previous
import functools

import jax
import jax.numpy as jnp
from jax import lax
from jax.experimental import pallas as pl
from jax.experimental.pallas import tpu as pltpu
from jax.experimental.pallas import tpu_sc as plsc

H = 2048
E = 16
TOPK = 2
DFF = 512
DSH = 512
T = 2048
BM = 256
NPAD = T * TOPK + E * BM
NB = NPAD // BM
NC = 2
NS = 16
NW = NC * NS
LANES = 16


def _silu(x):
    return x * jax.nn.sigmoid(x)


def _router_body(x_ref, gw_ref, logits_ref, d_ref, w_ref, tok_ref, blk_ref,
                 act_ref):
    x = x_ref[...]
    gw = gw_ref[...]
    logits = lax.dot_general(x, gw, (((1,), (1,)), ((), ())),
                             preferred_element_type=jnp.float32)
    logits_ref[...] = logits
    m = jnp.max(logits, axis=-1, keepdims=True)
    ex = jnp.exp(logits - m)
    probs = ex / jnp.sum(ex, axis=-1, keepdims=True)
    iota_e = lax.broadcasted_iota(jnp.int32, (T, E), 1)
    m1 = jnp.max(probs, axis=-1, keepdims=True)
    i1 = jnp.min(jnp.where(probs == m1, iota_e, E), axis=-1, keepdims=True)
    oh1 = iota_e == i1
    pm = jnp.where(oh1, -jnp.inf, probs)
    m2 = jnp.max(pm, axis=-1, keepdims=True)
    i2 = jnp.min(jnp.where(pm == m2, iota_e, E), axis=-1, keepdims=True)
    oh2 = iota_e == i2
    s = m1 + m2
    w_ref[...] = jnp.concatenate([m1 / s, m2 / s], axis=1)
    a = (oh1 | oh2).astype(jnp.float32)
    it_r = lax.broadcasted_iota(jnp.int32, (T, T), 0)
    it_c = lax.broadcasted_iota(jnp.int32, (T, T), 1)
    ltri = (it_c < it_r).astype(jnp.float32)
    rank = lax.dot_general(ltri, a, (((1,), (0,)), ((), ())),
                           preferred_element_type=jnp.float32)
    cnt = jnp.sum(a, axis=0, keepdims=True)
    cpad = jnp.ceil(cnt / BM) * BM
    ie_r = lax.broadcasted_iota(jnp.int32, (E, E), 0)
    ie_c = lax.broadcasted_iota(jnp.int32, (E, E), 1)
    utri = (ie_r < ie_c).astype(jnp.float32)
    off = lax.dot_general(cpad, utri, (((1,), (0,)), ((), ())),
                          preferred_element_type=jnp.float32)
    dst = off + rank
    d0 = jnp.sum(jnp.where(oh1, dst, 0.0), axis=1, keepdims=True)
    d1 = jnp.sum(jnp.where(oh2, dst, 0.0), axis=1, keepdims=True)
    d_ref[...] = jnp.concatenate([d0, d1], axis=1).astype(jnp.int32)
    tok_ref[...] = lax.broadcasted_iota(jnp.int32, (T, TOPK), 0)
    ones_t = jnp.ones((T, 1), jnp.float32)
    cnt_c = lax.dot_general(a, ones_t, (((0,), (0,)), ((), ())),
                            preferred_element_type=jnp.float32)
    cpad_c = jnp.ceil(cnt_c / BM) * BM
    ltri_e = (ie_c < ie_r).astype(jnp.float32)
    off_c = lax.dot_general(ltri_e, cpad_c, (((1,), (0,)), ((), ())),
                            preferred_element_type=jnp.float32)
    bpos = (lax.broadcasted_iota(jnp.int32, (1, NB), 1) * BM).astype(
        jnp.float32)
    blk = jnp.sum((off_c <= bpos).astype(jnp.int32), axis=0, keepdims=True) - 1
    blk_ref[...] = blk
    total = jnp.sum(cpad, axis=1, keepdims=True)
    act_ref[...] = (bpos < total).astype(jnp.int32)


def _router(x, gate_w):
    return pl.pallas_call(
        _router_body,
        out_shape=[
            jax.ShapeDtypeStruct((T, E), jnp.float32),
            jax.ShapeDtypeStruct((T, TOPK), jnp.int32),
            jax.ShapeDtypeStruct((T, TOPK), jnp.float32),
            jax.ShapeDtypeStruct((T, TOPK), jnp.int32),
            jax.ShapeDtypeStruct((1, NB), jnp.int32),
            jax.ShapeDtypeStruct((1, NB), jnp.int32),
        ],
    )(x, gate_w)


def _dispatch_body(x_hbm, tok_hbm, d_hbm, xs_hbm, tokv, dv, rowsa, rowsb,
                   gsem, ssem):
    wid = lax.axis_index("s") * NC + lax.axis_index("c")
    pltpu.sync_copy(tok_hbm.at[wid], tokv)
    pltpu.sync_copy(d_hbm.at[wid], dv)
    nch = T * TOPK // NW // LANES
    bufs = [rowsa, rowsb]
    gcp = [None] * nch
    scp = [None] * nch
    gcp[0] = pltpu.async_copy(x_hbm.at[tokv.at[0]], bufs[0], gsem)
    for ci in range(nch):
        cur = bufs[ci % 2]
        gcp[ci].wait()
        scp[ci] = pltpu.async_copy(cur, xs_hbm.at[dv.at[ci]], ssem)
        if ci + 1 < nch:
            if ci >= 1:
                scp[ci - 1].wait()
            gcp[ci + 1] = pltpu.async_copy(
                x_hbm.at[tokv.at[ci + 1]], bufs[(ci + 1) % 2], gsem)
    scp[nch - 2].wait()
    scp[nch - 1].wait()


def _dispatch(x, tok, d):
    nch = T * TOPK // NW // LANES
    mesh = plsc.VectorSubcoreMesh(core_axis_name="c", subcore_axis_name="s")
    kern = functools.partial(
        pl.kernel,
        mesh=mesh,
        out_type=jax.ShapeDtypeStruct((NPAD, H), jnp.float32),
        scratch_types=[
            pltpu.VMEM((nch, LANES), jnp.int32),
            pltpu.VMEM((nch, LANES), jnp.int32),
            pltpu.VMEM((LANES, H), jnp.float32),
            pltpu.VMEM((LANES, H), jnp.float32),
            pltpu.SemaphoreType.DMA,
            pltpu.SemaphoreType.DMA,
        ],
    )(_dispatch_body)
    return kern(x, tok.reshape(NW, nch, LANES), d.reshape(NW, nch, LANES))


def _gmm_body(blk_ref, act_ref, xs_ref, wg_ref, wu_ref, wd_ref, ys_ref):
    @pl.when(act_ref[0, pl.program_id(0)] == 1)
    def _():
        xb = xs_ref[...]
        hg = lax.dot_general(xb, wg_ref[0], (((1,), (1,)), ((), ())),
                             preferred_element_type=jnp.float32)
        hu = lax.dot_general(xb, wu_ref[0], (((1,), (1,)), ((), ())),
                             preferred_element_type=jnp.float32)
        hmid = _silu(hg) * hu
        ys_ref[...] = lax.dot_general(hmid, wd_ref[0],
                                      (((1,), (1,)), ((), ())),
                                      preferred_element_type=jnp.float32)


def _gmm(blk_e, act, xs, Wg, Wu, Wd):
    grid_spec = pltpu.PrefetchScalarGridSpec(
        num_scalar_prefetch=2,
        grid=(NB,),
        in_specs=[
            pl.BlockSpec((BM, H), lambda b, blk, act: (b, 0)),
            pl.BlockSpec((1, DFF, H), lambda b, blk, act: (blk[0, b], 0, 0)),
            pl.BlockSpec((1, DFF, H), lambda b, blk, act: (blk[0, b], 0, 0)),
            pl.BlockSpec((1, H, DFF), lambda b, blk, act: (blk[0, b], 0, 0)),
        ],
        out_specs=pl.BlockSpec((BM, H), lambda b, blk, act: (b, 0)),
    )
    return pl.pallas_call(
        _gmm_body,
        grid_spec=grid_spec,
        out_shape=jax.ShapeDtypeStruct((NPAD, H), jnp.float32),
    )(blk_e, act, xs, Wg, Wu, Wd)


def _shared_body(x_ref, sg_ref, su_ref, sd_ref, sgw_ref, out_ref):
    xb = x_ref[...]
    g = lax.dot_general(xb, sg_ref[...], (((1,), (1,)), ((), ())),
                        preferred_element_type=jnp.float32)
    u = lax.dot_general(xb, su_ref[...], (((1,), (1,)), ((), ())),
                        preferred_element_type=jnp.float32)
    mid = _silu(g) * u
    y = lax.dot_general(mid, sd_ref[...], (((1,), (1,)), ((), ())),
                        preferred_element_type=jnp.float32)
    gate = jax.nn.sigmoid(
        lax.dot_general(x_ref[...], sgw_ref[...], (((1,), (1,)), ((), ())),
                        preferred_element_type=jnp.float32))
    out_ref[...] = gate * y


def _shared(x, Sg, Su, Sd, sgw):
    nblk = T // BM
    return pl.pallas_call(
        _shared_body,
        grid=(nblk,),
        in_specs=[
            pl.BlockSpec((BM, H), lambda b: (b, 0)),
            pl.BlockSpec((DSH, H), lambda b: (0, 0)),
            pl.BlockSpec((DSH, H), lambda b: (0, 0)),
            pl.BlockSpec((H, DSH), lambda b: (0, 0)),
            pl.BlockSpec((1, H), lambda b: (0, 0)),
        ],
        out_specs=pl.BlockSpec((BM, H), lambda b: (b, 0)),
        out_shape=jax.ShapeDtypeStruct((T, H), jnp.float32),
    )(x, Sg, Su, Sd, sgw)


def _combine_body(ys_hbm, sh_hbm, d0_hbm, d1_hbm, w0_hbm, w1_hbm, out_hbm,
                  d0v, d1v, w0v, w1v, y0, y1, shv, sem):
    wid = lax.axis_index("s") * NC + lax.axis_index("c")
    tpw = T // NW
    nch = tpw // LANES
    base = wid * tpw
    pltpu.sync_copy(d0_hbm.at[wid], d0v)
    pltpu.sync_copy(d1_hbm.at[wid], d1v)
    pltpu.sync_copy(w0_hbm.at[wid], w0v)
    pltpu.sync_copy(w1_hbm.at[wid], w1v)
    for ci in range(nch):
        pltpu.async_copy(ys_hbm.at[d0v.at[ci]], y0, sem).wait()
        pltpu.async_copy(ys_hbm.at[d1v.at[ci]], y1, sem).wait()
        pltpu.sync_copy(sh_hbm.at[pl.ds(base + ci * LANES, LANES)], shv)
        w0c = w0v[ci]
        w1c = w1v[ci]
        dnums = lax.GatherDimensionNumbers(
            offset_dims=(), collapsed_slice_dims=(0,), start_index_map=(0,))
        for r in range(LANES):
            ridx = jnp.full((LANES, 1), r, jnp.int32)
            w0r = lax.gather(w0c, ridx, dnums, (1,),
                             mode=lax.GatherScatterMode.PROMISE_IN_BOUNDS)
            w1r = lax.gather(w1c, ridx, dnums, (1,),
                             mode=lax.GatherScatterMode.PROMISE_IN_BOUNDS)

            def col(cidx, carry, r=r, w0r=w0r, w1r=w1r):
                cs = pl.ds(cidx * LANES, LANES)
                shv[r, cs] = (w0r * y0[r, cs] + w1r * y1[r, cs] + shv[r, cs])
                return carry

            lax.fori_loop(0, H // LANES, col, 0)
        pltpu.sync_copy(shv, out_hbm.at[pl.ds(base + ci * LANES, LANES)])


def _combine(ys, sh, d, w):
    tpw = T // NW
    nch = tpw // LANES
    mesh = plsc.VectorSubcoreMesh(core_axis_name="c", subcore_axis_name="s")
    d0 = d[:, 0].reshape(NW, nch, LANES)
    d1 = d[:, 1].reshape(NW, nch, LANES)
    w0 = w[:, 0].reshape(NW, nch, LANES)
    w1 = w[:, 1].reshape(NW, nch, LANES)
    kern = functools.partial(
        pl.kernel,
        mesh=mesh,
        out_type=jax.ShapeDtypeStruct((T, H), jnp.float32),
        scratch_types=[
            pltpu.VMEM((nch, LANES), jnp.int32),
            pltpu.VMEM((nch, LANES), jnp.int32),
            pltpu.VMEM((nch, LANES), jnp.float32),
            pltpu.VMEM((nch, LANES), jnp.float32),
            pltpu.VMEM((LANES, H), jnp.float32),
            pltpu.VMEM((LANES, H), jnp.float32),
            pltpu.VMEM((LANES, H), jnp.float32),
            pltpu.SemaphoreType.DMA,
        ],
    )(_combine_body)
    return kern(ys, sh, d0, d1, w0, w1)


def kernel(hidden_states, gate_w, Wg, Wu, Wd, Sg, Su, Sd, shared_gate_w):
    b, s, h = hidden_states.shape
    x = hidden_states.reshape(-1, h)
    logits, d, w, tok, blk_e, act = _router(x, gate_w)
    xs = _dispatch(x, tok, d)
    ys = _gmm(blk_e, act, xs, Wg, Wu, Wd)
    sh = _shared(x, Sg, Su, Sd, shared_gate_w)
    final = _combine(ys, sh, d, w)
    return final.reshape(b, s, h), logits


# speedup vs baseline: 1.0638x; 1.0272x over previous
import functools

import jax
import jax.numpy as jnp
from jax import lax
from jax.experimental import pallas as pl
from jax.experimental.pallas import tpu as pltpu
from jax.experimental.pallas import tpu_sc as plsc

H = 2048
E = 16
TOPK = 2
DFF = 512
DSH = 512
T = 2048
BM = 256
NPAD = T * TOPK + E * BM
NB = NPAD // BM
NC = 2
NS = 16
NW = NC * NS
LANES = 16


def _silu(x):
    return x * jax.nn.sigmoid(x)


def _router_body(x_ref, gw_ref, logits_ref, d_ref, w_ref, tok_ref, blk_ref,
                 act_ref):
    x = x_ref[...]
    gw = gw_ref[...]
    logits = lax.dot_general(x, gw, (((1,), (1,)), ((), ())),
                             preferred_element_type=jnp.float32)
    logits_ref[...] = logits
    m = jnp.max(logits, axis=-1, keepdims=True)
    ex = jnp.exp(logits - m)
    probs = ex / jnp.sum(ex, axis=-1, keepdims=True)
    iota_e = lax.broadcasted_iota(jnp.int32, (T, E), 1)
    m1 = jnp.max(probs, axis=-1, keepdims=True)
    i1 = jnp.min(jnp.where(probs == m1, iota_e, E), axis=-1, keepdims=True)
    oh1 = iota_e == i1
    pm = jnp.where(oh1, -jnp.inf, probs)
    m2 = jnp.max(pm, axis=-1, keepdims=True)
    i2 = jnp.min(jnp.where(pm == m2, iota_e, E), axis=-1, keepdims=True)
    oh2 = iota_e == i2
    s = m1 + m2
    w_ref[...] = jnp.concatenate([m1 / s, m2 / s], axis=1)
    a = (oh1 | oh2).astype(jnp.float32)
    it_r = lax.broadcasted_iota(jnp.int32, (T, T), 0)
    it_c = lax.broadcasted_iota(jnp.int32, (T, T), 1)
    ltri = (it_c < it_r).astype(jnp.float32)
    rank = lax.dot_general(ltri, a, (((1,), (0,)), ((), ())),
                           preferred_element_type=jnp.float32)
    cnt = jnp.sum(a, axis=0, keepdims=True)
    cpad = jnp.ceil(cnt / BM) * BM
    ie_r = lax.broadcasted_iota(jnp.int32, (E, E), 0)
    ie_c = lax.broadcasted_iota(jnp.int32, (E, E), 1)
    utri = (ie_r < ie_c).astype(jnp.float32)
    off = lax.dot_general(cpad, utri, (((1,), (0,)), ((), ())),
                          preferred_element_type=jnp.float32)
    dst = off + rank
    d0 = jnp.sum(jnp.where(oh1, dst, 0.0), axis=1, keepdims=True)
    d1 = jnp.sum(jnp.where(oh2, dst, 0.0), axis=1, keepdims=True)
    d_ref[...] = jnp.concatenate([d0, d1], axis=1).astype(jnp.int32)
    tok_ref[...] = lax.broadcasted_iota(jnp.int32, (T, TOPK), 0)
    ones_t = jnp.ones((T, 1), jnp.float32)
    cnt_c = lax.dot_general(a, ones_t, (((0,), (0,)), ((), ())),
                            preferred_element_type=jnp.float32)
    cpad_c = jnp.ceil(cnt_c / BM) * BM
    ltri_e = (ie_c < ie_r).astype(jnp.float32)
    off_c = lax.dot_general(ltri_e, cpad_c, (((1,), (0,)), ((), ())),
                            preferred_element_type=jnp.float32)
    bpos = (lax.broadcasted_iota(jnp.int32, (1, NB), 1) * BM).astype(
        jnp.float32)
    blk = jnp.sum((off_c <= bpos).astype(jnp.int32), axis=0, keepdims=True) - 1
    blk_ref[...] = blk
    total = jnp.sum(cpad, axis=1, keepdims=True)
    act_ref[...] = (bpos < total).astype(jnp.int32)


def _router(x, gate_w):
    return pl.pallas_call(
        _router_body,
        out_shape=[
            jax.ShapeDtypeStruct((T, E), jnp.float32),
            jax.ShapeDtypeStruct((T, TOPK), jnp.int32),
            jax.ShapeDtypeStruct((T, TOPK), jnp.float32),
            jax.ShapeDtypeStruct((T, TOPK), jnp.int32),
            jax.ShapeDtypeStruct((1, NB), jnp.int32),
            jax.ShapeDtypeStruct((1, NB), jnp.int32),
        ],
    )(x, gate_w)


def _dispatch_body(x_hbm, tok_hbm, d_hbm, xs_hbm, tokv, dv, rowsa, rowsb,
                   gsem, ssem):
    wid = lax.axis_index("s") * NC + lax.axis_index("c")
    i0 = pltpu.async_copy(tok_hbm.at[wid], tokv, gsem)
    i1 = pltpu.async_copy(d_hbm.at[wid], dv, gsem)
    i0.wait()
    i1.wait()
    nch = T * TOPK // NW // LANES
    bufs = [rowsa, rowsb]
    gcp = [None] * nch
    scp = [None] * nch
    gcp[0] = pltpu.async_copy(x_hbm.at[tokv.at[0]], bufs[0], gsem)
    for ci in range(nch):
        cur = bufs[ci % 2]
        gcp[ci].wait()
        scp[ci] = pltpu.async_copy(cur, xs_hbm.at[dv.at[ci]], ssem)
        if ci + 1 < nch:
            if ci >= 1:
                scp[ci - 1].wait()
            gcp[ci + 1] = pltpu.async_copy(
                x_hbm.at[tokv.at[ci + 1]], bufs[(ci + 1) % 2], gsem)
    scp[nch - 2].wait()
    scp[nch - 1].wait()


def _dispatch(x, tok, d):
    nch = T * TOPK // NW // LANES
    mesh = plsc.VectorSubcoreMesh(core_axis_name="c", subcore_axis_name="s")
    kern = functools.partial(
        pl.kernel,
        mesh=mesh,
        out_type=jax.ShapeDtypeStruct((NPAD, H), jnp.float32),
        scratch_types=[
            pltpu.VMEM((nch, LANES), jnp.int32),
            pltpu.VMEM((nch, LANES), jnp.int32),
            pltpu.VMEM((LANES, H), jnp.float32),
            pltpu.VMEM((LANES, H), jnp.float32),
            pltpu.SemaphoreType.DMA,
            pltpu.SemaphoreType.DMA,
        ],
    )(_dispatch_body)
    return kern(x, tok.reshape(NW, nch, LANES), d.reshape(NW, nch, LANES))


def _gmm_body(blk_ref, act_ref, xs_ref, wg_ref, wu_ref, wd_ref, ys_ref):
    @pl.when(act_ref[0, pl.program_id(0)] == 1)
    def _():
        xb = xs_ref[...]
        hg = lax.dot_general(xb, wg_ref[0], (((1,), (1,)), ((), ())),
                             preferred_element_type=jnp.float32)
        hu = lax.dot_general(xb, wu_ref[0], (((1,), (1,)), ((), ())),
                             preferred_element_type=jnp.float32)
        hmid = _silu(hg) * hu
        ys_ref[...] = lax.dot_general(hmid, wd_ref[0],
                                      (((1,), (1,)), ((), ())),
                                      preferred_element_type=jnp.float32)


def _gmm(blk_e, act, xs, Wg, Wu, Wd):
    grid_spec = pltpu.PrefetchScalarGridSpec(
        num_scalar_prefetch=2,
        grid=(NB,),
        in_specs=[
            pl.BlockSpec((BM, H), lambda b, blk, act: (b, 0)),
            pl.BlockSpec((1, DFF, H), lambda b, blk, act: (blk[0, b], 0, 0)),
            pl.BlockSpec((1, DFF, H), lambda b, blk, act: (blk[0, b], 0, 0)),
            pl.BlockSpec((1, H, DFF), lambda b, blk, act: (blk[0, b], 0, 0)),
        ],
        out_specs=pl.BlockSpec((BM, H), lambda b, blk, act: (b, 0)),
    )
    return pl.pallas_call(
        _gmm_body,
        grid_spec=grid_spec,
        out_shape=jax.ShapeDtypeStruct((NPAD, H), jnp.float32),
    )(blk_e, act, xs, Wg, Wu, Wd)


def _shared_body(x_ref, sg_ref, su_ref, sd_ref, sgw_ref, out_ref):
    xb = x_ref[...]
    g = lax.dot_general(xb, sg_ref[...], (((1,), (1,)), ((), ())),
                        preferred_element_type=jnp.float32)
    u = lax.dot_general(xb, su_ref[...], (((1,), (1,)), ((), ())),
                        preferred_element_type=jnp.float32)
    mid = _silu(g) * u
    y = lax.dot_general(mid, sd_ref[...], (((1,), (1,)), ((), ())),
                        preferred_element_type=jnp.float32)
    gate = jax.nn.sigmoid(
        lax.dot_general(x_ref[...], sgw_ref[...], (((1,), (1,)), ((), ())),
                        preferred_element_type=jnp.float32))
    out_ref[...] = gate * y


def _shared(x, Sg, Su, Sd, sgw):
    nblk = T // BM
    return pl.pallas_call(
        _shared_body,
        grid=(nblk,),
        in_specs=[
            pl.BlockSpec((BM, H), lambda b: (b, 0)),
            pl.BlockSpec((DSH, H), lambda b: (0, 0)),
            pl.BlockSpec((DSH, H), lambda b: (0, 0)),
            pl.BlockSpec((H, DSH), lambda b: (0, 0)),
            pl.BlockSpec((1, H), lambda b: (0, 0)),
        ],
        out_specs=pl.BlockSpec((BM, H), lambda b: (b, 0)),
        out_shape=jax.ShapeDtypeStruct((T, H), jnp.float32),
    )(x, Sg, Su, Sd, sgw)


def _combine_body(ys_hbm, sh_hbm, d0_hbm, d1_hbm, w0_hbm, w1_hbm, out_hbm,
                  d0v, d1v, w0v, w1v, y0, y1, shv, sem):
    wid = lax.axis_index("s") * NC + lax.axis_index("c")
    tpw = T // NW
    nch = tpw // LANES
    base = wid * tpw
    m0 = pltpu.async_copy(d0_hbm.at[wid], d0v, sem)
    m1 = pltpu.async_copy(d1_hbm.at[wid], d1v, sem)
    m2 = pltpu.async_copy(w0_hbm.at[wid], w0v, sem)
    m3 = pltpu.async_copy(w1_hbm.at[wid], w1v, sem)
    m0.wait()
    m1.wait()
    m2.wait()
    m3.wait()
    for ci in range(nch):
        c0 = pltpu.async_copy(ys_hbm.at[d0v.at[ci]], y0, sem)
        c1 = pltpu.async_copy(ys_hbm.at[d1v.at[ci]], y1, sem)
        c2 = pltpu.async_copy(
            sh_hbm.at[pl.ds(base + ci * LANES, LANES)], shv, sem)
        c0.wait()
        c1.wait()
        c2.wait()
        w0c = w0v[ci]
        w1c = w1v[ci]
        dnums = lax.GatherDimensionNumbers(
            offset_dims=(), collapsed_slice_dims=(0,), start_index_map=(0,))
        for r in range(LANES):
            ridx = jnp.full((LANES, 1), r, jnp.int32)
            w0r = lax.gather(w0c, ridx, dnums, (1,),
                             mode=lax.GatherScatterMode.PROMISE_IN_BOUNDS)
            w1r = lax.gather(w1c, ridx, dnums, (1,),
                             mode=lax.GatherScatterMode.PROMISE_IN_BOUNDS)

            def col(cidx, carry, r=r, w0r=w0r, w1r=w1r):
                cs = pl.ds(cidx * LANES, LANES)
                shv[r, cs] = (w0r * y0[r, cs] + w1r * y1[r, cs] + shv[r, cs])
                return carry

            lax.fori_loop(0, H // LANES, col, 0)
        pltpu.sync_copy(shv, out_hbm.at[pl.ds(base + ci * LANES, LANES)])


def _combine(ys, sh, d, w):
    tpw = T // NW
    nch = tpw // LANES
    mesh = plsc.VectorSubcoreMesh(core_axis_name="c", subcore_axis_name="s")
    d0 = d[:, 0].reshape(NW, nch, LANES)
    d1 = d[:, 1].reshape(NW, nch, LANES)
    w0 = w[:, 0].reshape(NW, nch, LANES)
    w1 = w[:, 1].reshape(NW, nch, LANES)
    kern = functools.partial(
        pl.kernel,
        mesh=mesh,
        out_type=jax.ShapeDtypeStruct((T, H), jnp.float32),
        scratch_types=[
            pltpu.VMEM((nch, LANES), jnp.int32),
            pltpu.VMEM((nch, LANES), jnp.int32),
            pltpu.VMEM((nch, LANES), jnp.float32),
            pltpu.VMEM((nch, LANES), jnp.float32),
            pltpu.VMEM((LANES, H), jnp.float32),
            pltpu.VMEM((LANES, H), jnp.float32),
            pltpu.VMEM((LANES, H), jnp.float32),
            pltpu.SemaphoreType.DMA,
        ],
    )(_combine_body)
    return kern(ys, sh, d0, d1, w0, w1)


def kernel(hidden_states, gate_w, Wg, Wu, Wd, Sg, Su, Sd, shared_gate_w):
    b, s, h = hidden_states.shape
    x = hidden_states.reshape(-1, h)
    logits, d, w, tok, blk_e, act = _router(x, gate_w)
    xs = _dispatch(x, tok, d)
    ys = _gmm(blk_e, act, xs, Wg, Wu, Wd)
    sh = _shared(x, Sg, Su, Sd, shared_gate_w)
    final = _combine(ys, sh, d, w)
    return final.reshape(b, s, h), logits
